# baseline (speedup 1.0000x reference)
import functools

import jax
import jax.numpy as jnp
from jax import lax
from jax.experimental import pallas as pl
from jax.experimental.pallas import tpu as pltpu
from jax.experimental.pallas import tpu_sc as plsc

CH = 2048
CROWS = CH // 128
ZB = 128
BN = 2048

_mesh = plsc.VectorSubcoreMesh(core_axis_name="c", subcore_axis_name="s")


def _zero_fill(zbuf, d):
    zz = jnp.zeros((16,), jnp.float32)

    def zr(r, _):
        for k in range(d // 16):
            zbuf[r, pl.ds(16 * k, 16)] = zz
        return 0

    lax.fori_loop(0, ZB, zr, 0)


@functools.lru_cache(maxsize=None)
def _sc_norm(npad, ep):
    tpe = ep // 32
    nch = tpe // CH
    epr = ep // 128

    @functools.partial(
        pl.kernel, mesh=_mesh,
        compiler_params=pltpu.CompilerParams(use_tc_tiling_on_sc=False, needs_layout_passes=False),
        out_type=jax.ShapeDtypeStruct((epr, 128), jnp.float32),
        scratch_types=[
            pltpu.VMEM((npad // 128, 128), jnp.float32),
            pltpu.VMEM((CROWS, 3, 128), jnp.int32),
            pltpu.VMEM((CROWS, 128), jnp.float32),
        ],
    )
    def k(edata_hbm, dinv_hbm, out_hbm, dv, ebuf, nbuf):
        c = lax.axis_index("c")
        s = lax.axis_index("s")
        wid = c * 16 + s
        pltpu.sync_copy(dinv_hbm, dv)
        r0 = wid * (tpe // 128)

        def chunk(ci, _):
            pltpu.sync_copy(edata_hbm.at[pl.ds(r0 + ci * CROWS, CROWS)], ebuf)

            def grp(g, _):
                j = g >> 3
                q = (g & 7) * 16
                rv = ebuf[j, 0, pl.ds(q, 16)]
                cv = ebuf[j, 1, pl.ds(q, 16)]
                wv = plsc.bitcast(ebuf[j, 2, pl.ds(q, 16)], jnp.float32)
                a = plsc.load_gather(dv, [rv >> 7, rv & 127])
                b = plsc.load_gather(dv, [cv >> 7, cv & 127])
                nbuf[j, pl.ds(q, 16)] = a * b * wv
                return 0

            lax.fori_loop(0, CH // 16, grp, 0)
            pltpu.sync_copy(nbuf, out_hbm.at[pl.ds(r0 + ci * CROWS, CROWS)])
            return 0

        lax.fori_loop(0, nch, chunk, 0)

    return k


@functools.lru_cache(maxsize=None)
def _sc_agg(npad, ep, d, feature_split, ch, ones_z=False):
    if feature_split:
        tpe = ep // 16
    else:
        tpe = ep // 32
    nch = tpe // ch
    assert nch % 2 == 0 and nch >= 4
    crows = ch // 128
    rows_pt = npad // 16

    @functools.partial(
        pl.kernel, mesh=_mesh,
        compiler_params=pltpu.CompilerParams(use_tc_tiling_on_sc=False, needs_layout_passes=False),
        out_type=jax.ShapeDtypeStruct((2 * npad, d), jnp.float32),
        scratch_types=[
            pltpu.VMEM_SHARED((npad, d), jnp.float32),
            pltpu.VMEM((crows, 3, 128), jnp.int32),
            pltpu.VMEM((crows, 3, 128), jnp.int32),
            pltpu.VMEM((ch, d), jnp.float32),
            pltpu.VMEM((ch, d), jnp.float32),
            pltpu.VMEM((ZB, d), jnp.float32),
            pltpu.SemaphoreType.DMA,
            pltpu.SemaphoreType.DMA,
        ],
    )
    def k(z_hbm, edata_hbm, out_hbm,
          accum, ebuf0, ebuf1, rbuf0, rbuf1, zbuf, gsem, ssem):
        c = lax.axis_index("c")
        s = lax.axis_index("s")
        ebufs = (ebuf0, ebuf1)
        rbufs = (rbuf0, rbuf1)
        _zero_fill(zbuf, d)
        for kk in range(rows_pt // ZB):
            pltpu.sync_copy(zbuf, accum.at[pl.ds(s * rows_pt + kk * ZB, ZB)])
        plsc.subcore_barrier()

        if feature_split:
            r0 = s * (tpe // 128)
        else:
            r0 = (c * 16 + s) * (tpe // 128)

        def load_idx(q, b):
            pltpu.sync_copy(edata_hbm.at[pl.ds(r0 + q * crows, crows)],
                            ebufs[b])
            if feature_split:
                def remap(g, _):
                    j = g >> 3
                    qq = (g & 7) * 16
                    v = ebufs[b][j, 0, pl.ds(qq, 16)]
                    ebufs[b][j, 0, pl.ds(qq, 16)] = v * 2 + c
                    return 0

                lax.fori_loop(0, ch // 16, remap, 0)

        def fire_gathers(b):
            if ones_z:
                return
            for j in range(crows):
                pltpu.async_copy(z_hbm.at[ebufs[b].at[j, 0]],
                                 rbufs[b].at[pl.ds(j * 128, 128)], gsem)

        def drain(b, sem):
            if ones_z and sem is gsem:
                return
            pltpu.make_async_copy(z_hbm.at[pl.ds(0, ch)], rbufs[b], sem).wait()

        def scale(b):
            def body(gg, _):
                for u in range(2):
                    g = gg * 2 + u
                    j = g >> 3
                    q = (g & 7) * 16
                    nv = plsc.bitcast(
                        ebufs[b][j, 2, pl.ds(q, 16)], jnp.float32)
                    e0 = g * 16
                    for t in range(16):
                        bv = jnp.full((16,), nv[t], jnp.float32)
                        for kk in range(d // 16):
                            if ones_z:
                                rbufs[b][e0 + t, pl.ds(16 * kk, 16)] = bv
                            else:
                                rbufs[b][e0 + t, pl.ds(16 * kk, 16)] = (
                                    rbufs[b][e0 + t, pl.ds(16 * kk, 16)] * bv)
                return 0

            lax.fori_loop(0, ch // 32, body, 0)

        def fire_scatters(b):
            for j in range(crows):
                pltpu.async_copy(rbufs[b].at[pl.ds(j * 128, 128)],
                                 accum.at[ebufs[b].at[j, 1]], ssem, add=True)

        load_idx(0, 0)
        fire_gathers(0)
        load_idx(1, 1)
        fire_gathers(1)
        drain(0, gsem)
        scale(0)
        fire_scatters(0)

        def pair(pi, _):
            for b in (1, 0):
                q = 2 * pi + (1 if b == 1 else 2)
                ob = 1 - b
                drain(ob, ssem)
                load_idx(q + 1, ob)
                fire_gathers(ob)
                drain(b, gsem)
                scale(b)
                fire_scatters(b)
            return 0

        lax.fori_loop(0, (nch - 2) // 2, pair, 0)

        drain(0, ssem)
        drain(1, gsem)
        scale(1)
        fire_scatters(1)
        drain(1, ssem)

        plsc.subcore_barrier()
        pltpu.sync_copy(
            accum.at[pl.ds(s * rows_pt, rows_pt)],
            out_hbm.at[pl.ds(c * npad + s * rows_pt, rows_pt)])

    return k


def _tc_prep(parts):
    npad = parts.shape[0] // 2
    nb = npad // BN

    def body(p0, p1, o_ref):
        deg = jnp.sum(p0[...] + p1[...], axis=1, keepdims=True) * (1.0 / 16.0)
        o_ref[...] = jnp.broadcast_to(lax.rsqrt(deg), (BN, 16))

    return pl.pallas_call(
        body, grid=(nb,),
        in_specs=[
            pl.BlockSpec((BN, 16), lambda i: (i, 0)),
            pl.BlockSpec((BN, 16), lambda i, _nb=nb: (i + _nb, 0)),
        ],
        out_specs=pl.BlockSpec((BN, 16), lambda i: (i, 0)),
        out_shape=jax.ShapeDtypeStruct((npad, 16), jnp.float32),
    )(parts, parts)


def _tc_layer1(sx, aW1p, ab1, cW1p, cb1):
    n = sx.shape[0] // 2
    nb = n // BN

    def body(s0, s1, wa, ba, wc, bc, oh, oc):
        agg = s0[...] + s1[...]
        oh[...] = jax.nn.sigmoid(
            jnp.dot(agg, wa[...], preferred_element_type=jnp.float32) + ba[...])
        oc[...] = jax.nn.relu(
            jnp.dot(agg, wc[...], preferred_element_type=jnp.float32) + bc[...])

    return pl.pallas_call(
        body,
        grid=(nb,),
        in_specs=[
            pl.BlockSpec((BN, 16), lambda i: (i, 0)),
            pl.BlockSpec((BN, 16), lambda i, _nb=nb: (i + _nb, 0)),
            pl.BlockSpec((16, 64), lambda i: (0, 0)),
            pl.BlockSpec((1, 64), lambda i: (0, 0)),
            pl.BlockSpec((16, 16), lambda i: (0, 0)),
            pl.BlockSpec((1, 16), lambda i: (0, 0)),
        ],
        out_specs=[
            pl.BlockSpec((BN, 64), lambda i: (i, 0)),
            pl.BlockSpec((BN, 16), lambda i: (i, 0)),
        ],
        out_shape=[
            jax.ShapeDtypeStruct((n, 64), jnp.float32),
            jax.ShapeDtypeStruct((n, 16), jnp.float32),
        ],
    )(sx, sx, aW1p, ab1.reshape(1, 64), cW1p, cb1.reshape(1, 16))


def _tc_layer64(s64, w, b, head_w=None, head_b=None):
    n = s64.shape[0] // 2
    nb = n // BN
    with_head = head_w is not None

    def body(s0, s1, w_ref, b_ref, *rest):
        agg = jnp.concatenate([s0[...], s1[...]], axis=1)
        h = jax.nn.sigmoid(
            jnp.dot(agg, w_ref[...], preferred_element_type=jnp.float32)
            + b_ref[...])
        if with_head:
            hw, hb, o_ref = rest
            o_ref[...] = (
                jnp.dot(h, hw[...], preferred_element_type=jnp.float32)
                + hb[...])
        else:
            (o_ref,) = rest
            o_ref[...] = h

    in_specs = [
        pl.BlockSpec((BN, 32), lambda i: (i, 0)),
        pl.BlockSpec((BN, 32), lambda i, _nb=nb: (i + _nb, 0)),
        pl.BlockSpec((64, 64), lambda i: (0, 0)),
        pl.BlockSpec((1, 64), lambda i: (0, 0)),
    ]
    args = [s64, s64, w, b.reshape(1, 64)]
    if with_head:
        in_specs += [
            pl.BlockSpec((64, 1), lambda i: (0, 0)),
            pl.BlockSpec((1, 1), lambda i: (0, 0)),
        ]
        args += [head_w, head_b.reshape(1, 1)]
        out_spec = pl.BlockSpec((BN, 1), lambda i: (i, 0))
        out_shape = jax.ShapeDtypeStruct((n, 1), jnp.float32)
    else:
        out_spec = pl.BlockSpec((BN, 64), lambda i: (i, 0))
        out_shape = jax.ShapeDtypeStruct((n, 64), jnp.float32)

    return pl.pallas_call(
        body, grid=(nb,), in_specs=in_specs,
        out_specs=out_spec, out_shape=out_shape,
    )(*args)


def _tc_layer16(s16, w, b):
    n = s16.shape[0] // 2
    nb = n // BN

    def body(s0, s1, w_ref, b_ref, o_ref):
        agg = s0[...] + s1[...]
        o_ref[...] = jax.nn.relu(
            jnp.dot(agg, w_ref[...], preferred_element_type=jnp.float32)
            + b_ref[...])

    return pl.pallas_call(
        body, grid=(nb,),
        in_specs=[
            pl.BlockSpec((BN, 16), lambda i: (i, 0)),
            pl.BlockSpec((BN, 16), lambda i, _nb=nb: (i + _nb, 0)),
            pl.BlockSpec((16, 16), lambda i: (0, 0)),
            pl.BlockSpec((1, 16), lambda i: (0, 0)),
        ],
        out_specs=pl.BlockSpec((BN, 16), lambda i: (i, 0)),
        out_shape=jax.ShapeDtypeStruct((n, 16), jnp.float32),
    )(s16, s16, w, b.reshape(1, 16))


def _tc_critic_head(s16, w, b, nreal):
    npad = s16.shape[0] // 2
    nb = npad // BN

    def body(s0, s1, w_ref, b_ref, o_ref):
        i = pl.program_id(0)
        agg = s0[...] + s1[...]
        c4 = jax.nn.relu(
            jnp.dot(agg, w_ref[...], preferred_element_type=jnp.float32)
            + b_ref[...])
        rid = i * BN + lax.broadcasted_iota(jnp.int32, (BN, 1), 0)
        c4 = jnp.where(rid < nreal, c4, 0.0)
        part = jnp.sum(c4, keepdims=True).reshape(1, 1)
        prev = jnp.where(i == 0, jnp.zeros((1, 1), jnp.float32), o_ref[...])
        tot = prev + part
        o_ref[...] = jnp.where(i == nb - 1, tot / nreal, tot)

    return pl.pallas_call(
        body, grid=(nb,),
        in_specs=[
            pl.BlockSpec((BN, 16), lambda i: (i, 0)),
            pl.BlockSpec((BN, 16), lambda i, _nb=nb: (i + _nb, 0)),
            pl.BlockSpec((16, 1), lambda i: (0, 0)),
            pl.BlockSpec((1, 1), lambda i: (0, 0)),
        ],
        out_specs=pl.BlockSpec((1, 1), lambda i: (0, 0)),
        out_shape=jax.ShapeDtypeStruct((1, 1), jnp.float32),
    )(s16, s16, w, b.reshape(1, 1))


def _tc_softmax(e_r):
    def body(e_ref, o_ref):
        e = e_ref[...]
        m = jnp.max(e)
        p = jnp.exp(e - m)
        o_ref[...] = p / jnp.sum(p)

    return pl.pallas_call(
        body,
        out_shape=jax.ShapeDtypeStruct(e_r.shape, jnp.float32),
    )(e_r)


def kernel(vertex_embeddings, edges, weights, aW1, ab1, aW2, ab2, aW3, ab3,
           alW, alb, cW1, cb1, cW2, cb2, cW3, cb3, cW4, cb4):
    x = vertex_embeddings.astype(jnp.float32)
    n = x.shape[0]
    e = weights.shape[0]
    row = edges[0].astype(jnp.int32)
    col = edges[1].astype(jnp.int32)
    loop = jnp.arange(n, dtype=jnp.int32)
    unit = 32 * CH
    ep = ((e + n + unit - 1) // unit) * unit
    pad = ep - e - n
    epr = ep // 128

    npad = -(-n // BN) * BN

    rowf = jnp.concatenate(
        [row, loop, jnp.zeros((pad,), jnp.int32)]).reshape(epr, 128)
    colf = jnp.concatenate(
        [col, loop, jnp.zeros((pad,), jnp.int32)]).reshape(epr, 128)
    wf = jnp.concatenate(
        [weights.astype(jnp.float32), jnp.ones((n,), jnp.float32),
         jnp.zeros((pad,), jnp.float32)]).reshape(epr, 128)

    agg16 = _sc_agg(npad, ep, 16, False, 1024)
    agg64 = _sc_agg(npad, ep, 32, True, 256)
    agg_deg = _sc_agg(npad, ep, 16, False, 1024, True)

    edata_w = jnp.stack(
        [rowf, colf, lax.bitcast_convert_type(wf, jnp.int32)], axis=1)
    ones16 = jnp.ones((npad, 16), jnp.float32)
    parts = agg_deg(ones16, edata_w)
    dinv16 = _tc_prep(parts)
    dinv_rl = dinv16[:, :1].reshape(npad // 128, 128)
    norm2 = _sc_norm(npad, ep)(edata_w, dinv_rl)
    edata = jnp.stack(
        [rowf, colf, lax.bitcast_convert_type(norm2, jnp.int32)], axis=1)

    xp = jnp.pad(x, ((0, npad - n), (0, 16 - x.shape[1])))
    aW1p = jnp.zeros((16, 64), jnp.float32).at[:6, :].set(aW1)
    cW1p = jnp.zeros((16, 16), jnp.float32).at[:6, :].set(cW1)

    sx = agg16(xp, edata)
    h1, c1 = _tc_layer1(sx, aW1p, ab1, cW1p, cb1)

    sh1 = agg64(h1.reshape(2 * npad, 32), edata)
    h2 = _tc_layer64(sh1, aW2, ab2)
    sh2 = agg64(h2.reshape(2 * npad, 32), edata)
    e_nodes = _tc_layer64(sh2, aW3, ab3, head_w=alW, head_b=alb)
    policy = _tc_softmax(e_nodes[:n].reshape(400, 125)).reshape(n, 1)

    sc1 = agg16(c1, edata)
    c2 = _tc_layer16(sc1, cW2, cb2)
    sc2 = agg16(c2, edata)
    c3 = _tc_layer16(sc2, cW3, cb3)
    sc3 = agg16(c3, edata)
    value = _tc_critic_head(sc3, cW4, cb4, n)

    return (policy, value)


# speedup vs baseline: 1.4133x; 1.4133x over previous
import functools

import jax
import jax.numpy as jnp
from jax import lax
from jax.experimental import pallas as pl
from jax.experimental.pallas import tpu as pltpu
from jax.experimental.pallas import tpu_sc as plsc

CH = 2048
CROWS = CH // 128
ZB = 128
BN = 2048

_mesh = plsc.VectorSubcoreMesh(core_axis_name="c", subcore_axis_name="s")


def _zero_fill(zbuf, d):
    zz = jnp.zeros((16,), jnp.float32)

    def zr(r, _):
        for k in range(d // 16):
            zbuf[r, pl.ds(16 * k, 16)] = zz
        return 0

    lax.fori_loop(0, ZB, zr, 0)


@functools.lru_cache(maxsize=None)
def _sc_norm(npad, ep):
    tpe = ep // 32
    nch = tpe // CH
    epr = ep // 128

    @functools.partial(
        pl.kernel, mesh=_mesh,
        compiler_params=pltpu.CompilerParams(use_tc_tiling_on_sc=False, needs_layout_passes=False),
        out_type=jax.ShapeDtypeStruct((epr, 128), jnp.float32),
        scratch_types=[
            pltpu.VMEM((npad // 128, 128), jnp.float32),
            pltpu.VMEM((CROWS, 3, 128), jnp.int32),
            pltpu.VMEM((CROWS, 128), jnp.float32),
        ],
    )
    def k(edata_hbm, dinv_hbm, out_hbm, dv, ebuf, nbuf):
        c = lax.axis_index("c")
        s = lax.axis_index("s")
        wid = c * 16 + s
        pltpu.sync_copy(dinv_hbm, dv)
        r0 = wid * (tpe // 128)

        def chunk(ci, _):
            pltpu.sync_copy(edata_hbm.at[pl.ds(r0 + ci * CROWS, CROWS)], ebuf)

            def grp(g, _):
                j = g >> 3
                q = (g & 7) * 16
                rv = ebuf[j, 0, pl.ds(q, 16)]
                cv = ebuf[j, 1, pl.ds(q, 16)]
                wv = plsc.bitcast(ebuf[j, 2, pl.ds(q, 16)], jnp.float32)
                a = plsc.load_gather(dv, [rv >> 7, rv & 127])
                b = plsc.load_gather(dv, [cv >> 7, cv & 127])
                nbuf[j, pl.ds(q, 16)] = a * b * wv
                return 0

            lax.fori_loop(0, CH // 16, grp, 0)
            pltpu.sync_copy(nbuf, out_hbm.at[pl.ds(r0 + ci * CROWS, CROWS)])
            return 0

        lax.fori_loop(0, nch, chunk, 0)

    return k


@functools.lru_cache(maxsize=None)
def _sc_agg(npad, ep, d, feature_split, ch, ones_z=False):
    if feature_split:
        tpe = ep // 16
    else:
        tpe = ep // 32
    nch = tpe // ch
    assert nch % 2 == 0 and nch >= 4
    crows = ch // 128
    rows_pt = npad // 16

    @functools.partial(
        pl.kernel, mesh=_mesh,
        compiler_params=pltpu.CompilerParams(use_tc_tiling_on_sc=False, needs_layout_passes=False),
        out_type=jax.ShapeDtypeStruct((2 * npad, d), jnp.float32),
        scratch_types=[
            pltpu.VMEM_SHARED((npad, d), jnp.float32),
            pltpu.VMEM((crows, 3, 128), jnp.int32),
            pltpu.VMEM((crows, 3, 128), jnp.int32),
            pltpu.VMEM((ch, d), jnp.float32),
            pltpu.VMEM((ch, d), jnp.float32),
            pltpu.VMEM((ZB, d), jnp.float32),
            pltpu.SemaphoreType.DMA,
            pltpu.SemaphoreType.DMA,
        ],
    )
    def k(z_hbm, edata_hbm, out_hbm,
          accum, ebuf0, ebuf1, rbuf0, rbuf1, zbuf, gsem, ssem):
        c = lax.axis_index("c")
        s = lax.axis_index("s")
        ebufs = (ebuf0, ebuf1)
        rbufs = (rbuf0, rbuf1)
        _zero_fill(zbuf, d)
        for kk in range(rows_pt // ZB):
            pltpu.sync_copy(zbuf, accum.at[pl.ds(s * rows_pt + kk * ZB, ZB)])
        plsc.subcore_barrier()

        if feature_split:
            r0 = s * (tpe // 128)
        else:
            r0 = (c * 16 + s) * (tpe // 128)

        def load_idx(q, b):
            pltpu.sync_copy(edata_hbm.at[pl.ds(r0 + q * crows, crows)],
                            ebufs[b])
            if feature_split:
                def remap(g, _):
                    j = g >> 3
                    qq = (g & 7) * 16
                    v = ebufs[b][j, 0, pl.ds(qq, 16)]
                    ebufs[b][j, 0, pl.ds(qq, 16)] = v * 2 + c
                    return 0

                lax.fori_loop(0, ch // 16, remap, 0)

        def fire_gathers(b):
            if ones_z:
                return
            for j in range(crows):
                pltpu.async_copy(z_hbm.at[ebufs[b].at[j, 0]],
                                 rbufs[b].at[pl.ds(j * 128, 128)], gsem)

        def drain(b, sem):
            if ones_z and sem is gsem:
                return
            pltpu.make_async_copy(z_hbm.at[pl.ds(0, ch)], rbufs[b], sem).wait()

        def scale(b):
            def body(g, _):
                j = g >> 3
                q = (g & 7) * 16
                nv = plsc.bitcast(ebufs[b][j, 2, pl.ds(q, 16)], jnp.float32)
                e0 = g * 16
                for t in range(16):
                    bv = jnp.full((16,), nv[t], jnp.float32)
                    for kk in range(d // 16):
                        if ones_z:
                            rbufs[b][e0 + t, pl.ds(16 * kk, 16)] = bv
                        else:
                            rbufs[b][e0 + t, pl.ds(16 * kk, 16)] = (
                                rbufs[b][e0 + t, pl.ds(16 * kk, 16)] * bv)
                return 0

            lax.fori_loop(0, ch // 16, body, 0)

        def fire_scatters(b):
            for j in range(crows):
                pltpu.async_copy(rbufs[b].at[pl.ds(j * 128, 128)],
                                 accum.at[ebufs[b].at[j, 1]], ssem, add=True)

        load_idx(0, 0)
        fire_gathers(0)
        load_idx(1, 1)
        fire_gathers(1)
        drain(0, gsem)
        scale(0)
        fire_scatters(0)

        def pair(pi, _):
            for b in (1, 0):
                q = 2 * pi + (1 if b == 1 else 2)
                ob = 1 - b
                drain(ob, ssem)
                load_idx(q + 1, ob)
                fire_gathers(ob)
                drain(b, gsem)
                scale(b)
                fire_scatters(b)
            return 0

        lax.fori_loop(0, (nch - 2) // 2, pair, 0)

        drain(0, ssem)
        drain(1, gsem)
        scale(1)
        fire_scatters(1)
        drain(1, ssem)

        plsc.subcore_barrier()
        pltpu.sync_copy(
            accum.at[pl.ds(s * rows_pt, rows_pt)],
            out_hbm.at[pl.ds(c * npad + s * rows_pt, rows_pt)])

    return k


def _tc_prep(parts):
    npad = parts.shape[0] // 2
    nb = npad // BN

    def body(p0, p1, o_ref):
        deg = jnp.sum(p0[...] + p1[...], axis=1, keepdims=True) * (1.0 / 16.0)
        o_ref[...] = jnp.broadcast_to(lax.rsqrt(deg), (BN, 16))

    return pl.pallas_call(
        body, grid=(nb,),
        in_specs=[
            pl.BlockSpec((BN, 16), lambda i: (i, 0)),
            pl.BlockSpec((BN, 16), lambda i, _nb=nb: (i + _nb, 0)),
        ],
        out_specs=pl.BlockSpec((BN, 16), lambda i: (i, 0)),
        out_shape=jax.ShapeDtypeStruct((npad, 16), jnp.float32),
    )(parts, parts)


def _tc_layer1(sx, aW1p, ab1, cW1p, cb1):
    n = sx.shape[0] // 2
    nb = n // BN

    def body(s0, s1, wa, ba, wc, bc, oh, oc):
        agg = s0[...] + s1[...]
        oh[...] = jax.nn.sigmoid(
            jnp.dot(agg, wa[...], preferred_element_type=jnp.float32) + ba[...])
        oc[...] = jax.nn.relu(
            jnp.dot(agg, wc[...], preferred_element_type=jnp.float32) + bc[...])

    return pl.pallas_call(
        body,
        grid=(nb,),
        in_specs=[
            pl.BlockSpec((BN, 16), lambda i: (i, 0)),
            pl.BlockSpec((BN, 16), lambda i, _nb=nb: (i + _nb, 0)),
            pl.BlockSpec((16, 64), lambda i: (0, 0)),
            pl.BlockSpec((1, 64), lambda i: (0, 0)),
            pl.BlockSpec((16, 16), lambda i: (0, 0)),
            pl.BlockSpec((1, 16), lambda i: (0, 0)),
        ],
        out_specs=[
            pl.BlockSpec((BN, 64), lambda i: (i, 0)),
            pl.BlockSpec((BN, 16), lambda i: (i, 0)),
        ],
        out_shape=[
            jax.ShapeDtypeStruct((n, 64), jnp.float32),
            jax.ShapeDtypeStruct((n, 16), jnp.float32),
        ],
    )(sx, sx, aW1p, ab1.reshape(1, 64), cW1p, cb1.reshape(1, 16))


def _tc_layer64(s64, w, b, head_w=None, head_b=None):
    n = s64.shape[0] // 2
    nb = n // BN
    with_head = head_w is not None

    def body(s0, s1, w_ref, b_ref, *rest):
        agg = jnp.concatenate([s0[...], s1[...]], axis=1)
        h = jax.nn.sigmoid(
            jnp.dot(agg, w_ref[...], preferred_element_type=jnp.float32)
            + b_ref[...])
        if with_head:
            hw, hb, o_ref = rest
            o_ref[...] = (
                jnp.dot(h, hw[...], preferred_element_type=jnp.float32)
                + hb[...])
        else:
            (o_ref,) = rest
            o_ref[...] = h

    in_specs = [
        pl.BlockSpec((BN, 32), lambda i: (i, 0)),
        pl.BlockSpec((BN, 32), lambda i, _nb=nb: (i + _nb, 0)),
        pl.BlockSpec((64, 64), lambda i: (0, 0)),
        pl.BlockSpec((1, 64), lambda i: (0, 0)),
    ]
    args = [s64, s64, w, b.reshape(1, 64)]
    if with_head:
        in_specs += [
            pl.BlockSpec((64, 1), lambda i: (0, 0)),
            pl.BlockSpec((1, 1), lambda i: (0, 0)),
        ]
        args += [head_w, head_b.reshape(1, 1)]
        out_spec = pl.BlockSpec((BN, 1), lambda i: (i, 0))
        out_shape = jax.ShapeDtypeStruct((n, 1), jnp.float32)
    else:
        out_spec = pl.BlockSpec((BN, 64), lambda i: (i, 0))
        out_shape = jax.ShapeDtypeStruct((n, 64), jnp.float32)

    return pl.pallas_call(
        body, grid=(nb,), in_specs=in_specs,
        out_specs=out_spec, out_shape=out_shape,
    )(*args)


def _tc_layer16(s16, w, b):
    n = s16.shape[0] // 2
    nb = n // BN

    def body(s0, s1, w_ref, b_ref, o_ref):
        agg = s0[...] + s1[...]
        o_ref[...] = jax.nn.relu(
            jnp.dot(agg, w_ref[...], preferred_element_type=jnp.float32)
            + b_ref[...])

    return pl.pallas_call(
        body, grid=(nb,),
        in_specs=[
            pl.BlockSpec((BN, 16), lambda i: (i, 0)),
            pl.BlockSpec((BN, 16), lambda i, _nb=nb: (i + _nb, 0)),
            pl.BlockSpec((16, 16), lambda i: (0, 0)),
            pl.BlockSpec((1, 16), lambda i: (0, 0)),
        ],
        out_specs=pl.BlockSpec((BN, 16), lambda i: (i, 0)),
        out_shape=jax.ShapeDtypeStruct((n, 16), jnp.float32),
    )(s16, s16, w, b.reshape(1, 16))


def _tc_critic_head(s16, w, b, nreal):
    npad = s16.shape[0] // 2
    nb = npad // BN

    def body(s0, s1, w_ref, b_ref, o_ref):
        i = pl.program_id(0)
        agg = s0[...] + s1[...]
        c4 = jax.nn.relu(
            jnp.dot(agg, w_ref[...], preferred_element_type=jnp.float32)
            + b_ref[...])
        rid = i * BN + lax.broadcasted_iota(jnp.int32, (BN, 1), 0)
        c4 = jnp.where(rid < nreal, c4, 0.0)
        part = jnp.sum(c4, keepdims=True).reshape(1, 1)
        prev = jnp.where(i == 0, jnp.zeros((1, 1), jnp.float32), o_ref[...])
        tot = prev + part
        o_ref[...] = jnp.where(i == nb - 1, tot / nreal, tot)

    return pl.pallas_call(
        body, grid=(nb,),
        in_specs=[
            pl.BlockSpec((BN, 16), lambda i: (i, 0)),
            pl.BlockSpec((BN, 16), lambda i, _nb=nb: (i + _nb, 0)),
            pl.BlockSpec((16, 1), lambda i: (0, 0)),
            pl.BlockSpec((1, 1), lambda i: (0, 0)),
        ],
        out_specs=pl.BlockSpec((1, 1), lambda i: (0, 0)),
        out_shape=jax.ShapeDtypeStruct((1, 1), jnp.float32),
    )(s16, s16, w, b.reshape(1, 1))


def _tc_softmax(e_r):
    def body(e_ref, o_ref):
        e = e_ref[...]
        m = jnp.max(e)
        p = jnp.exp(e - m)
        o_ref[...] = p / jnp.sum(p)

    return pl.pallas_call(
        body,
        out_shape=jax.ShapeDtypeStruct(e_r.shape, jnp.float32),
    )(e_r)


def kernel(vertex_embeddings, edges, weights, aW1, ab1, aW2, ab2, aW3, ab3,
           alW, alb, cW1, cb1, cW2, cb2, cW3, cb3, cW4, cb4):
    x = vertex_embeddings.astype(jnp.float32)
    n = x.shape[0]
    e = weights.shape[0]
    row = edges[0].astype(jnp.int32)
    col = edges[1].astype(jnp.int32)
    loop = jnp.arange(n, dtype=jnp.int32)
    unit = 32 * CH
    ep = ((e + n + unit - 1) // unit) * unit
    pad = ep - e - n
    epr = ep // 128

    npad = -(-n // BN) * BN

    rowf = jnp.concatenate(
        [row, loop, jnp.zeros((pad,), jnp.int32)]).reshape(epr, 128)
    colf = jnp.concatenate(
        [col, loop, jnp.zeros((pad,), jnp.int32)]).reshape(epr, 128)
    wf = jnp.concatenate(
        [weights.astype(jnp.float32), jnp.ones((n,), jnp.float32),
         jnp.zeros((pad,), jnp.float32)]).reshape(epr, 128)

    agg16 = _sc_agg(npad, ep, 16, False, 1024)
    agg64 = _sc_agg(npad, ep, 32, True, 256)
    agg_deg = _sc_agg(npad, ep, 16, False, 1024, True)

    edata_w = jnp.stack(
        [rowf, colf, lax.bitcast_convert_type(wf, jnp.int32)], axis=1)
    ones16 = jnp.ones((npad, 16), jnp.float32)
    parts = agg_deg(ones16, edata_w)
    dinv16 = _tc_prep(parts)
    dinv_rl = dinv16[:, :1].reshape(npad // 128, 128)
    norm2 = _sc_norm(npad, ep)(edata_w, dinv_rl)
    edata = jnp.stack(
        [rowf, colf, lax.bitcast_convert_type(norm2, jnp.int32)], axis=1)

    xp = jnp.pad(x, ((0, npad - n), (0, 16 - x.shape[1])))
    aW1p = jnp.zeros((16, 64), jnp.float32).at[:6, :].set(aW1)
    cW1p = jnp.zeros((16, 16), jnp.float32).at[:6, :].set(cW1)

    sx = agg16(xp, edata)
    h1, c1 = _tc_layer1(sx, aW1p, ab1, cW1p, cb1)

    sh1 = agg64(h1.reshape(2 * npad, 32), edata)
    h2 = _tc_layer64(sh1, aW2, ab2)
    sh2 = agg64(h2.reshape(2 * npad, 32), edata)
    e_nodes = _tc_layer64(sh2, aW3, ab3, head_w=alW, head_b=alb)
    policy = _tc_softmax(e_nodes[:n].reshape(400, 125)).reshape(n, 1)

    sc1 = agg16(c1, edata)
    c2 = _tc_layer16(sc1, cW2, cb2)
    sc2 = agg16(c2, edata)
    c3 = _tc_layer16(sc2, cW3, cb3)
    sc3 = agg16(c3, edata)
    value = _tc_critic_head(sc3, cW4, cb4, n)

    return (policy, value)
